# baseline (speedup 1.0000x reference)
import functools

import jax
import jax.numpy as jnp
from jax import lax
from jax.experimental import pallas as pl
from jax.experimental.pallas import tpu as pltpu
from jax.experimental.pallas import tpu_sc as plsc

_DIM = 1000
_B0 = 1024
_B1 = 26
_N_ROWS = _B0 * _B1
_N_ELEMS = _N_ROWS * _DIM
_NC = 2
_NS = 16
_NW = _NC * _NS
_ROWS_PER_W = _N_ROWS // _NW
_SCAT_ROWS = 7
_SCAT_W = 128
_GROUPS = _ROWS_PER_W // 16

_ZBLK = 2048 * 1000
_ZGRID = _N_ELEMS // _ZBLK
_NSEM = 4


@functools.partial(
    pl.kernel,
    out_type=(),
    mesh=pltpu.create_tensorcore_mesh("x"),
    scratch_types=[
        pltpu.VMEM((_ZBLK,), jnp.float32),
        pltpu.SemaphoreType.DMA((_NSEM,)),
    ],
)
def _zero_fill_tc(out_ref, zbuf, sems):
    zbuf[...] = jnp.zeros((_ZBLK,), jnp.float32)
    copies = [
        pltpu.async_copy(
            zbuf, out_ref.at[pl.ds(k * _ZBLK, _ZBLK)], sems.at[k % _NSEM]
        )
        for k in range(_ZGRID)
    ]
    for cp in copies:
        cp.wait()


@functools.partial(
    pl.kernel,
    out_type=(),
    mesh=plsc.VectorSubcoreMesh(core_axis_name="c", subcore_axis_name="s"),
    compiler_params=pltpu.CompilerParams(
        use_tc_tiling_on_sc=False, needs_layout_passes=False
    ),
    scratch_types=[
        pltpu.VMEM((_ROWS_PER_W,), jnp.int32),
        pltpu.VMEM((_SCAT_ROWS, _SCAT_W), jnp.int32),
        pltpu.VMEM((_SCAT_W,), jnp.float32),
        pltpu.SemaphoreType.DMA((_NSEM,)),
    ],
)
def _scatter_ones_sc(idx_hbm, out_ref, idx_v, pos_v, ones_v, sems):
    wid = lax.axis_index("s") * _NC + lax.axis_index("c")
    base = wid * _ROWS_PER_W

    pltpu.sync_copy(idx_hbm.at[pl.ds(base, _ROWS_PER_W)], idx_v)

    lane = lax.iota(jnp.int32, 16)
    ones = jnp.ones((16,), jnp.float32)
    for c in range(_SCAT_W // 16):
        ones_v[pl.ds(c * 16, 16)] = ones
    for g in range(_SCAT_ROWS * _SCAT_W // 16):
        sg = g if g < _GROUPS else g - 4
        off = sg * 16
        idx16 = idx_v[pl.ds(off, 16)]
        r = base + off + lane
        i = r // _B1
        j = r - i * _B1
        pos = (
            j * (_DIM * _B0)
            + (idx16 >> 3) * (8 * _B0)
            + (i >> 7) * (8 * 128)
            + (idx16 & 7) * 128
            + (i & 127)
        )
        pos_v[g // 8, pl.ds((g % 8) * 16, 16)] = pos
    copies = [
        pltpu.async_copy(
            ones_v, out_ref.at[pos_v.at[jr]], sems.at[jr % _NSEM]
        )
        for jr in range(_SCAT_ROWS)
    ]
    for cp in copies:
        cp.wait()


def kernel(tensor):
    idx = tensor.reshape(_N_ROWS).astype(jnp.int32)
    flat = jax.empty_ref(jax.ShapeDtypeStruct((_N_ELEMS,), jnp.float32))
    _zero_fill_tc(flat)
    _scatter_ones_sc(idx, flat)
    out6 = flat[...].reshape(_B1, _DIM // 8, _B0 // 128, 8, 128)
    return out6.transpose(2, 4, 0, 1, 3).reshape(_B0, _B1, _DIM)


# speedup vs baseline: 1.0423x; 1.0423x over previous
import functools

import jax
import jax.numpy as jnp
from jax import lax
from jax.experimental import pallas as pl
from jax.experimental.pallas import tpu as pltpu
from jax.experimental.pallas import tpu_sc as plsc

_DIM = 1000
_B0 = 1024
_B1 = 26
_N_ROWS = _B0 * _B1
_N_ELEMS = _N_ROWS * _DIM
_NC = 2
_NS = 16
_NW = _NC * _NS
_ROWS_PER_W = _N_ROWS // _NW
_SCAT_ROWS = 13
_SCAT_W = 64
_GROUPS = _ROWS_PER_W // 16

_ZBLK = 1024 * 1000
_ZGRID = _N_ELEMS // _ZBLK
_NSEM = 8


@functools.partial(
    pl.kernel,
    out_type=(),
    mesh=pltpu.create_tensorcore_mesh("x"),
    scratch_types=[
        pltpu.VMEM((_ZBLK,), jnp.float32),
        pltpu.VMEM((_ZBLK,), jnp.float32),
        pltpu.SemaphoreType.DMA((_NSEM,)),
    ],
)
def _zero_fill_tc(out_ref, zbuf_a, zbuf_b, sems):
    zbuf_a[...] = jnp.zeros((_ZBLK,), jnp.float32)
    zbuf_b[...] = jnp.zeros((_ZBLK,), jnp.float32)
    bufs = (zbuf_a, zbuf_b)
    copies = [
        pltpu.async_copy(
            bufs[k % 2], out_ref.at[pl.ds(k * _ZBLK, _ZBLK)], sems.at[k % _NSEM]
        )
        for k in range(_ZGRID)
    ]
    for cp in copies:
        cp.wait()


@functools.partial(
    pl.kernel,
    out_type=(),
    mesh=plsc.VectorSubcoreMesh(core_axis_name="c", subcore_axis_name="s"),
    compiler_params=pltpu.CompilerParams(
        use_tc_tiling_on_sc=False, needs_layout_passes=False
    ),
    scratch_types=[
        pltpu.VMEM((_ROWS_PER_W,), jnp.int32),
        pltpu.VMEM((_SCAT_ROWS, _SCAT_W), jnp.int32),
        pltpu.VMEM((_SCAT_W,), jnp.float32),
        pltpu.SemaphoreType.DMA((_NSEM,)),
    ],
)
def _scatter_ones_sc(idx_hbm, out_ref, idx_v, pos_v, ones_v, sems):
    wid = lax.axis_index("s") * _NC + lax.axis_index("c")
    base = wid * _ROWS_PER_W

    pltpu.sync_copy(idx_hbm.at[pl.ds(base, _ROWS_PER_W)], idx_v)

    lane = lax.iota(jnp.int32, 16)
    ones = jnp.ones((16,), jnp.float32)
    for c in range(_SCAT_W // 16):
        ones_v[pl.ds(c * 16, 16)] = ones
    for g in range(_GROUPS):
        off = g * 16
        idx16 = idx_v[pl.ds(off, 16)]
        r = base + off + lane
        i = r // _B1
        j = r - i * _B1
        pos = (
            j * (_DIM * _B0)
            + (idx16 >> 3) * (8 * _B0)
            + (i >> 7) * (8 * 128)
            + (idx16 & 7) * 128
            + (i & 127)
        )
        pos_v[g // 4, pl.ds((g % 4) * 16, 16)] = pos
    copies = [
        pltpu.async_copy(
            ones_v, out_ref.at[pos_v.at[jr]], sems.at[jr % _NSEM]
        )
        for jr in range(_SCAT_ROWS)
    ]
    for cp in copies:
        cp.wait()


def kernel(tensor):
    idx = tensor.reshape(_N_ROWS).astype(jnp.int32)
    flat = jax.empty_ref(jax.ShapeDtypeStruct((_N_ELEMS,), jnp.float32))
    _zero_fill_tc(flat)
    _scatter_ones_sc(idx, flat)
    out6 = flat[...].reshape(_B1, _DIM // 8, _B0 // 128, 8, 128)
    return out6.transpose(2, 4, 0, 1, 3).reshape(_B0, _B1, _DIM)
